# baseline (speedup 1.0000x reference)
import jax
import jax.numpy as jnp
from jax import lax
from jax.experimental import pallas as pl
from jax.experimental.pallas import tpu as pltpu
from jax.experimental.pallas import tpu_sc as plsc

NUM_FIXED = 2
D = 64
BATCH = 16384
SEQ = 26
NC, NS, L = 2, 16, 16
NW = NC * NS

TBL = 999998
W = 384
NBLK = 999936 // W
TMAIN = NBLK * W
NEXTRA = NUM_FIXED + (TBL - TMAIN)

B_PER_W = BATCH // NW
SB = 32
N_SUP = B_PER_W // SB
NGRP = SB * SEQ // L


H = TMAIN // 2
KB = 5376
NTB = H // KB


def _tr_body(a_ref, b_ref, o_ref):
    r = lax.broadcasted_iota(jnp.int32, (D, 2 * D), 0)
    c = lax.broadcasted_iota(jnp.int32, (D, 2 * D), 1)
    ea = (c == r).astype(jnp.float32)
    eb = (c == r + D).astype(jnp.float32)
    dn = (((0,), (0,)), ((), ()))
    o_ref[...] = (
        lax.dot_general(a_ref[...], ea, dn, precision=lax.Precision.HIGHEST)
        + lax.dot_general(b_ref[...], eb, dn, precision=lax.Precision.HIGHEST))


def _gbody(idx_hbm, extras_hbm, train_hbm, out_hbm,
           idx_v, idxc0, idxc1, rows0, rows1, extras_v, gsem, osem0, osem1):
    wid = lax.axis_index("s") * NC + lax.axis_index("c")
    idxcs = (idxc0, idxc1)
    rows = (rows0, rows1)
    osems = (osem0, osem1)
    pltpu.sync_copy(extras_hbm, extras_v)

    def superchunk(s2, carry):
        for par in range(2):
            s = s2 * 2 + par
            b0 = wid * B_PER_W + s * SB
            rows_v = rows[par]
            idxc_v = idxcs[par]

            @pl.when(s >= 2)
            def _():
                pltpu.make_async_copy(
                    rows_v, out_hbm.at[pl.ds(0, SB)], osems[par]).wait()

            pltpu.sync_copy(idx_hbm.at[pl.ds(b0, SB)], idx_v)

            def prep(g, c):
                p = g * L + lax.iota(jnp.int32, L)
                r = p // SEQ
                col = p % SEQ
                iv = plsc.load_gather(idx_v, [r, col])
                t = jnp.clip(iv - NUM_FIXED, 0, TMAIN - 1)
                f = jnp.where(t >= H, 2 * (t - H) + 1, 2 * t)
                plsc.store_scatter(idxc_v, [r, col], f)
                return c

            lax.fori_loop(0, NGRP, prep, 0)

            cps = [
                pltpu.async_copy(
                    train_hbm.at[idxc_v.at[bb]], rows_v.at[bb], gsem)
                for bb in range(SB)
            ]
            for cp in cps:
                cp.wait()

            def fix(g, c):
                p = g * L + lax.iota(jnp.int32, L)
                r = p // SEQ
                col = p % SEQ
                iv = plsc.load_gather(idx_v, [r, col])
                m_lo = iv < NUM_FIXED
                m_hi = iv >= TMAIN + NUM_FIXED
                m = m_lo | m_hi

                @pl.when(plsc.all_reduce_population_count(m)[0] > 0)
                def _():
                    e = jnp.where(m_lo, iv, iv - TMAIN)
                    e = jnp.clip(e, 0, NEXTRA - 1)
                    for cc in range(D):
                        cvec = jnp.full((L,), cc, jnp.int32)
                        v = plsc.load_gather(extras_v, [e, cvec], mask=m)
                        plsc.store_scatter(rows_v, [r, col, cvec], v, mask=m)

                return c

            lax.fori_loop(0, NGRP, fix, 0)

            pltpu.async_copy(rows_v, out_hbm.at[pl.ds(b0, SB)], osems[par])

        return carry

    lax.fori_loop(0, N_SUP // 2, superchunk, 0)

    for par in range(2):
        pltpu.make_async_copy(
            rows[par], out_hbm.at[pl.ds(0, SB)], osems[par]).wait()


@jax.jit
def _run(idx, weights_freeze, weights_train):
    mesh = plsc.VectorSubcoreMesh(core_axis_name="c", subcore_axis_name="s")

    transpose = pl.pallas_call(
        _tr_body,
        grid=(NTB,),
        in_specs=[
            pl.BlockSpec((D, KB), lambda i: (0, i)),
            pl.BlockSpec((D, KB), lambda i: (0, i + NTB)),
        ],
        out_specs=pl.BlockSpec((KB, 2 * D), lambda i: (i, 0)),
        out_shape=jax.ShapeDtypeStruct((H, 2 * D), jnp.float32),
    )
    wt = weights_train.T
    table2d = transpose(wt, wt)
    table_lin = table2d.reshape(TMAIN, D)
    extras = jnp.concatenate(
        [weights_freeze, weights_train[TMAIN:]], axis=0)

    gather = pl.kernel(
        _gbody,
        out_type=jax.ShapeDtypeStruct((BATCH, SEQ, D), jnp.float32),
        mesh=mesh,
        scratch_types=[
            pltpu.VMEM((SB, SEQ), jnp.int32),
            pltpu.VMEM((SB, SEQ), jnp.int32),
            pltpu.VMEM((SB, SEQ), jnp.int32),
            pltpu.VMEM((SB, SEQ, D), jnp.float32),
            pltpu.VMEM((SB, SEQ, D), jnp.float32),
            pltpu.VMEM((NEXTRA, D), jnp.float32),
            pltpu.SemaphoreType.DMA,
            pltpu.SemaphoreType.DMA,
            pltpu.SemaphoreType.DMA,
        ],
        compiler_params=pltpu.CompilerParams(
            needs_layout_passes=False, use_tc_tiling_on_sc=False),
    )
    return gather(idx, extras, table_lin)


def kernel(idx, weights_freeze, weights_train):
    return _run(idx.astype(jnp.int32), weights_freeze.astype(jnp.float32),
                weights_train.astype(jnp.float32))


# speedup vs baseline: 1.4598x; 1.4598x over previous
import jax
import jax.numpy as jnp
from jax import lax
from jax.experimental import pallas as pl
from jax.experimental.pallas import tpu as pltpu
from jax.experimental.pallas import tpu_sc as plsc

NUM_FIXED = 2
D = 64
BATCH = 16384
SEQ = 26
NC, NS, L = 2, 16, 16
NW = NC * NS

TBL = 999998
W = 384
NBLK = 999936 // W
TMAIN = NBLK * W
NEXTRA = NUM_FIXED + (TBL - TMAIN)

B_PER_W = BATCH // NW
SB = 32
N_SUP = B_PER_W // SB
NGRP = SB * SEQ // L


H = TMAIN // 2
KB = 11904
NTB = H // KB


def _tr_body(a_ref, b_ref, o_ref):
    o_ref[:, 0:D] = a_ref[...].T
    o_ref[:, D:2 * D] = b_ref[...].T


def _gbody(idx_hbm, extras_hbm, train_hbm, out_hbm,
           idx_v, idxc0, idxc1, rows0, rows1, extras_v, gsem, osem0, osem1):
    wid = lax.axis_index("s") * NC + lax.axis_index("c")
    idxcs = (idxc0, idxc1)
    rows = (rows0, rows1)
    osems = (osem0, osem1)
    pltpu.sync_copy(extras_hbm, extras_v)

    def superchunk(s2, carry):
        for par in range(2):
            s = s2 * 2 + par
            b0 = wid * B_PER_W + s * SB
            rows_v = rows[par]
            idxc_v = idxcs[par]

            @pl.when(s >= 2)
            def _():
                pltpu.make_async_copy(
                    rows_v, out_hbm.at[pl.ds(0, SB)], osems[par]).wait()

            pltpu.sync_copy(idx_hbm.at[pl.ds(b0, SB)], idx_v)

            def prep(g, c):
                p = g * L + lax.iota(jnp.int32, L)
                r = p // SEQ
                col = p % SEQ
                iv = plsc.load_gather(idx_v, [r, col])
                t = jnp.clip(iv - NUM_FIXED, 0, TMAIN - 1)
                f = jnp.where(t >= H, 2 * (t - H) + 1, 2 * t)
                plsc.store_scatter(idxc_v, [r, col], f)
                return c

            lax.fori_loop(0, NGRP, prep, 0)

            cps = [
                pltpu.async_copy(
                    train_hbm.at[idxc_v.at[bb]], rows_v.at[bb], gsem)
                for bb in range(SB)
            ]
            for cp in cps:
                cp.wait()

            def fix(g, c):
                p = g * L + lax.iota(jnp.int32, L)
                r = p // SEQ
                col = p % SEQ
                iv = plsc.load_gather(idx_v, [r, col])
                m_lo = iv < NUM_FIXED
                m_hi = iv >= TMAIN + NUM_FIXED
                m = m_lo | m_hi

                @pl.when(plsc.all_reduce_population_count(m)[0] > 0)
                def _():
                    e = jnp.where(m_lo, iv, iv - TMAIN)
                    e = jnp.clip(e, 0, NEXTRA - 1)
                    for cc in range(D):
                        cvec = jnp.full((L,), cc, jnp.int32)
                        v = plsc.load_gather(extras_v, [e, cvec], mask=m)
                        plsc.store_scatter(rows_v, [r, col, cvec], v, mask=m)

                return c

            lax.fori_loop(0, NGRP, fix, 0)

            pltpu.async_copy(rows_v, out_hbm.at[pl.ds(b0, SB)], osems[par])

        return carry

    lax.fori_loop(0, N_SUP // 2, superchunk, 0)

    for par in range(2):
        pltpu.make_async_copy(
            rows[par], out_hbm.at[pl.ds(0, SB)], osems[par]).wait()


@jax.jit
def _run(idx, weights_freeze, weights_train):
    mesh = plsc.VectorSubcoreMesh(core_axis_name="c", subcore_axis_name="s")

    transpose = pl.pallas_call(
        _tr_body,
        grid=(NTB,),
        in_specs=[
            pl.BlockSpec((D, KB), lambda i: (0, i)),
            pl.BlockSpec((D, KB), lambda i: (0, i + NTB)),
        ],
        out_specs=pl.BlockSpec((KB, 2 * D), lambda i: (i, 0)),
        out_shape=jax.ShapeDtypeStruct((H, 2 * D), jnp.float32),
    )
    wt = weights_train.T
    table2d = transpose(wt, wt)
    table_lin = table2d.reshape(TMAIN, D)
    extras = jnp.concatenate(
        [weights_freeze, weights_train[TMAIN:]], axis=0)

    gather = pl.kernel(
        _gbody,
        out_type=jax.ShapeDtypeStruct((BATCH, SEQ, D), jnp.float32),
        mesh=mesh,
        scratch_types=[
            pltpu.VMEM((SB, SEQ), jnp.int32),
            pltpu.VMEM((SB, SEQ), jnp.int32),
            pltpu.VMEM((SB, SEQ), jnp.int32),
            pltpu.VMEM((SB, SEQ, D), jnp.float32),
            pltpu.VMEM((SB, SEQ, D), jnp.float32),
            pltpu.VMEM((NEXTRA, D), jnp.float32),
            pltpu.SemaphoreType.DMA,
            pltpu.SemaphoreType.DMA,
            pltpu.SemaphoreType.DMA,
        ],
        compiler_params=pltpu.CompilerParams(
            needs_layout_passes=False, use_tc_tiling_on_sc=False),
    )
    return gather(idx, extras, table_lin)


def kernel(idx, weights_freeze, weights_train):
    return _run(idx.astype(jnp.int32), weights_freeze.astype(jnp.float32),
                weights_train.astype(jnp.float32))
